# baseline (speedup 1.0000x reference)
import jax
import jax.numpy as jnp
from jax import lax
from jax.experimental import pallas as pl
from jax.experimental.pallas import tpu as pltpu

NUM_HEADS = 16
HEAD_DIM = 128
NUM_KV_HEADS = 4
GROUP = NUM_HEADS // NUM_KV_HEADS
SCALE = 0.08838834764831845

SEQ = 2048
BQ = 256
BK = 256


def _flash_kernel(q_ref, k_ref, v_ref, o_ref):
    i = pl.program_id(0)
    n_blocks = (i + 1) * (BQ // BK)
    row = i * BQ + lax.broadcasted_iota(jnp.int32, (BQ, BK), 0)

    for h in range(NUM_HEADS):
        g = h // GROUP
        q = q_ref[:, h * HEAD_DIM:(h + 1) * HEAD_DIM] * SCALE

        def body(j, carry, g=g, q=q):
            m, l, acc = carry
            k_blk = k_ref[pl.ds(j * BK, BK), g * HEAD_DIM:(g + 1) * HEAD_DIM]
            s = lax.dot_general(
                q, k_blk, (((1,), (1,)), ((), ())),
                preferred_element_type=jnp.float32,
            )
            col = j * BK + lax.broadcasted_iota(jnp.int32, (BQ, BK), 1)
            s = jnp.where(row >= col, s, -jnp.inf)
            m_new = jnp.maximum(m, jnp.max(s, axis=1, keepdims=True))
            p = jnp.exp(s - m_new)
            alpha = jnp.exp(m - m_new)
            v_blk = v_ref[pl.ds(j * BK, BK), g * HEAD_DIM:(g + 1) * HEAD_DIM]
            pv = lax.dot_general(
                p, v_blk, (((1,), (0,)), ((), ())),
                preferred_element_type=jnp.float32,
            )
            l = l * alpha + jnp.sum(p, axis=1, keepdims=True)
            acc = acc * alpha + pv
            return m_new, l, acc

        m0 = jnp.full((BQ, 1), -jnp.inf, jnp.float32)
        l0 = jnp.zeros((BQ, 1), jnp.float32)
        acc0 = jnp.zeros((BQ, HEAD_DIM), jnp.float32)
        m, l, acc = lax.fori_loop(0, n_blocks, body, (m0, l0, acc0))
        o_ref[:, h * HEAD_DIM:(h + 1) * HEAD_DIM] = acc / l


@jax.jit
def _attention(q2, k2, v2):
    return pl.pallas_call(
        _flash_kernel,
        grid=(SEQ // BQ,),
        in_specs=[
            pl.BlockSpec((BQ, NUM_HEADS * HEAD_DIM), lambda i: (i, 0)),
            pl.BlockSpec((SEQ, NUM_KV_HEADS * HEAD_DIM), lambda i: (0, 0)),
            pl.BlockSpec((SEQ, NUM_KV_HEADS * HEAD_DIM), lambda i: (0, 0)),
        ],
        out_specs=pl.BlockSpec((BQ, NUM_HEADS * HEAD_DIM), lambda i: (i, 0)),
        out_shape=jax.ShapeDtypeStruct((SEQ, NUM_HEADS * HEAD_DIM), jnp.float32),
        compiler_params=pltpu.CompilerParams(
            dimension_semantics=("arbitrary",),
        ),
    )(q2, k2, v2)


def kernel(q, k, v, cu_seqlens_q):
    q2 = q.reshape(SEQ, NUM_HEADS * HEAD_DIM)
    k2 = k.reshape(SEQ, NUM_KV_HEADS * HEAD_DIM)
    v2 = v.reshape(SEQ, NUM_KV_HEADS * HEAD_DIM)
    return _attention(q2, k2, v2)
